# baseline (speedup 1.0000x reference)
import jax
import jax.numpy as jnp
from jax.experimental import pallas as pl
from jax.experimental.pallas import tpu as pltpu

_N = 4096
_F = 512
_BM = 256
_BP = 512


def _premix_body(x_ref, s_ref, war_ref, wma_ref, xw_ref, sw_ref):
    xw_ref[...] = jnp.dot(
        x_ref[...].astype(jnp.bfloat16), war_ref[...],
        preferred_element_type=jnp.float32).astype(jnp.bfloat16)
    sw_ref[...] = jnp.dot(
        s_ref[...].astype(jnp.bfloat16), wma_ref[...],
        preferred_element_type=jnp.float32).astype(jnp.bfloat16)


def _main_body(ar_ref, ma_ref, xw_ref, sw_ref, b_ref, o_ref):
    acc = jnp.dot(ar_ref[...].astype(jnp.bfloat16), xw_ref[...],
                  preferred_element_type=jnp.float32)
    acc = acc + jnp.dot(ma_ref[...].astype(jnp.bfloat16), sw_ref[...],
                        preferred_element_type=jnp.float32)
    o_ref[...] = jnp.maximum(acc + b_ref[...], 0.0)


def kernel(x, arma_conv_AR, arma_conv_MA, input_signal, ar_kernel, ma_kernel, bias):
    n, f_in = x.shape
    f_out = ar_kernel.shape[1]

    war16 = ar_kernel.astype(jnp.bfloat16)
    wma16 = ma_kernel.astype(jnp.bfloat16)

    xw, sw = pl.pallas_call(
        _premix_body,
        grid=(n // _BP,),
        in_specs=[
            pl.BlockSpec((_BP, f_in), lambda i: (i, 0)),
            pl.BlockSpec((_BP, f_in), lambda i: (i, 0)),
            pl.BlockSpec((f_in, f_out), lambda i: (0, 0)),
            pl.BlockSpec((f_in, f_out), lambda i: (0, 0)),
        ],
        out_specs=[
            pl.BlockSpec((_BP, f_out), lambda i: (i, 0)),
            pl.BlockSpec((_BP, f_out), lambda i: (i, 0)),
        ],
        out_shape=[
            jax.ShapeDtypeStruct((n, f_out), jnp.bfloat16),
            jax.ShapeDtypeStruct((n, f_out), jnp.bfloat16),
        ],
        compiler_params=pltpu.CompilerParams(
            dimension_semantics=("parallel",)),
    )(x, input_signal, war16, wma16)

    out = pl.pallas_call(
        _main_body,
        grid=(n // _BM,),
        in_specs=[
            pl.BlockSpec((_BM, n), lambda i: (i, 0)),
            pl.BlockSpec((_BM, n), lambda i: (i, 0)),
            pl.BlockSpec((n, f_out), lambda i: (0, 0)),
            pl.BlockSpec((n, f_out), lambda i: (0, 0)),
            pl.BlockSpec((1, f_out), lambda i: (0, 0)),
        ],
        out_specs=pl.BlockSpec((_BM, f_out), lambda i: (i, 0)),
        out_shape=jax.ShapeDtypeStruct((n, f_out), jnp.float32),
        compiler_params=pltpu.CompilerParams(
            dimension_semantics=("parallel",)),
    )(arma_conv_AR, arma_conv_MA, xw, sw, bias.reshape(1, f_out))

    return out


# speedup vs baseline: 1.0416x; 1.0416x over previous
import jax
import jax.numpy as jnp
from jax.experimental import pallas as pl
from jax.experimental.pallas import tpu as pltpu

_N = 4096
_F = 512
_BM = 512
_BP = 512


def _premix_body(x_ref, s_ref, war_ref, wma_ref, xw_ref, sw_ref):
    xw_ref[...] = jnp.dot(
        x_ref[...].astype(jnp.bfloat16), war_ref[...],
        preferred_element_type=jnp.float32).astype(jnp.bfloat16)
    sw_ref[...] = jnp.dot(
        s_ref[...].astype(jnp.bfloat16), wma_ref[...],
        preferred_element_type=jnp.float32).astype(jnp.bfloat16)


def _main_body(ar_ref, ma_ref, xw_ref, sw_ref, b_ref, o_ref):
    acc = jnp.dot(ar_ref[...].astype(jnp.bfloat16), xw_ref[...],
                  preferred_element_type=jnp.float32)
    acc = acc + jnp.dot(ma_ref[...].astype(jnp.bfloat16), sw_ref[...],
                        preferred_element_type=jnp.float32)
    o_ref[...] = jnp.maximum(acc + b_ref[...], 0.0)


def kernel(x, arma_conv_AR, arma_conv_MA, input_signal, ar_kernel, ma_kernel, bias):
    n, f_in = x.shape
    f_out = ar_kernel.shape[1]

    war16 = ar_kernel.astype(jnp.bfloat16)
    wma16 = ma_kernel.astype(jnp.bfloat16)

    xw, sw = pl.pallas_call(
        _premix_body,
        grid=(n // _BP,),
        in_specs=[
            pl.BlockSpec((_BP, f_in), lambda i: (i, 0)),
            pl.BlockSpec((_BP, f_in), lambda i: (i, 0)),
            pl.BlockSpec((f_in, f_out), lambda i: (0, 0)),
            pl.BlockSpec((f_in, f_out), lambda i: (0, 0)),
        ],
        out_specs=[
            pl.BlockSpec((_BP, f_out), lambda i: (i, 0)),
            pl.BlockSpec((_BP, f_out), lambda i: (i, 0)),
        ],
        out_shape=[
            jax.ShapeDtypeStruct((n, f_out), jnp.bfloat16),
            jax.ShapeDtypeStruct((n, f_out), jnp.bfloat16),
        ],
        compiler_params=pltpu.CompilerParams(
            dimension_semantics=("parallel",)),
    )(x, input_signal, war16, wma16)

    out = pl.pallas_call(
        _main_body,
        grid=(n // _BM,),
        in_specs=[
            pl.BlockSpec((_BM, n), lambda i: (i, 0)),
            pl.BlockSpec((_BM, n), lambda i: (i, 0)),
            pl.BlockSpec((n, f_out), lambda i: (0, 0)),
            pl.BlockSpec((n, f_out), lambda i: (0, 0)),
            pl.BlockSpec((1, f_out), lambda i: (0, 0)),
        ],
        out_specs=pl.BlockSpec((_BM, f_out), lambda i: (i, 0)),
        out_shape=jax.ShapeDtypeStruct((n, f_out), jnp.float32),
        compiler_params=pltpu.CompilerParams(
            dimension_semantics=("parallel",)),
    )(arma_conv_AR, arma_conv_MA, xw, sw, bias.reshape(1, f_out))

    return out


# speedup vs baseline: 1.1234x; 1.0786x over previous
import jax
import jax.numpy as jnp
from jax.experimental import pallas as pl
from jax.experimental.pallas import tpu as pltpu

_BK = 512


def _body(x_ref, s_ref, war_ref, wma_ref, ar_ref, ma_ref, b_ref, o_ref):
    k = pl.program_id(0)
    xw = jnp.dot(x_ref[...].astype(jnp.bfloat16), war_ref[...],
                 preferred_element_type=jnp.float32).astype(jnp.bfloat16)
    sw = jnp.dot(s_ref[...].astype(jnp.bfloat16), wma_ref[...],
                 preferred_element_type=jnp.float32).astype(jnp.bfloat16)
    part = jnp.dot(ar_ref[...].astype(jnp.bfloat16), xw,
                   preferred_element_type=jnp.float32)
    part = part + jnp.dot(ma_ref[...].astype(jnp.bfloat16), sw,
                          preferred_element_type=jnp.float32)

    @pl.when(k == 0)
    def _init():
        o_ref[...] = part + b_ref[...]

    @pl.when(k > 0)
    def _acc():
        o_ref[...] += part

    @pl.when(k == pl.num_programs(0) - 1)
    def _fin():
        o_ref[...] = jnp.maximum(o_ref[...], 0.0)


def kernel(x, arma_conv_AR, arma_conv_MA, input_signal, ar_kernel, ma_kernel, bias):
    n, f_in = x.shape
    f_out = ar_kernel.shape[1]

    war16 = ar_kernel.astype(jnp.bfloat16)
    wma16 = ma_kernel.astype(jnp.bfloat16)

    out = pl.pallas_call(
        _body,
        grid=(n // _BK,),
        in_specs=[
            pl.BlockSpec((_BK, f_in), lambda k: (k, 0)),
            pl.BlockSpec((_BK, f_in), lambda k: (k, 0)),
            pl.BlockSpec((f_in, f_out), lambda k: (0, 0)),
            pl.BlockSpec((f_in, f_out), lambda k: (0, 0)),
            pl.BlockSpec((n, _BK), lambda k: (0, k)),
            pl.BlockSpec((n, _BK), lambda k: (0, k)),
            pl.BlockSpec((1, f_out), lambda k: (0, 0)),
        ],
        out_specs=pl.BlockSpec((n, f_out), lambda k: (0, 0)),
        out_shape=jax.ShapeDtypeStruct((n, f_out), jnp.float32),
        compiler_params=pltpu.CompilerParams(
            dimension_semantics=("arbitrary",)),
    )(x, input_signal, war16, wma16, arma_conv_AR, arma_conv_MA,
      bias.reshape(1, f_out))

    return out
